# Initial kernel scaffold; baseline (speedup 1.0000x reference)
import jax
import jax.numpy as jnp
from jax.experimental import pallas as pl


def kernel(x, no0, no1, W1, b1, g1, be1, W2, b2, g2, be2, W3, b3, g3, be3, W4, b4, g4, be4, W5, b5, g5, be5, Wfc, bfc):
    raise NotImplementedError("write your pallas kernel here")


# speedup vs baseline: 5.0097x; 5.0097x over previous
import functools

import jax
import jax.numpy as jnp
from jax import lax
from jax.experimental import pallas as pl
from jax.experimental.pallas import tpu as pltpu
from jax.experimental.pallas import tpu_sc as plsc

N0 = 163842
N1 = 40962
NW = 32
B0 = 128
B1 = 64
RW0 = 5248
RW1 = 1344
N0P = NW * RW0
N1P = NW * RW1
BN = 2048
EPS = 1e-5


def _tc_y_plain_body(h_ref, w_ref, out_ref):
    out_ref[...] = lax.dot_general(
        h_ref[...], w_ref[...], (((1,), (1,)), ((), ())),
        preferred_element_type=jnp.float32)


def _tc_y_plain(h, w_all, n_pad, c_in, c_out7):
    nb = n_pad // BN
    return pl.pallas_call(
        _tc_y_plain_body,
        grid=(nb,),
        in_specs=[
            pl.BlockSpec((BN, c_in), lambda i: (i, 0)),
            pl.BlockSpec(w_all.shape, lambda i: (0, 0)),
        ],
        out_specs=pl.BlockSpec((BN, c_out7), lambda i: (i, 0)),
        out_shape=jax.ShapeDtypeStruct((n_pad, c_out7), jnp.float32),
    )(h, w_all)


def _tc_y_norm_body(n_true, z_ref, st_ref, g_ref, be_ref, w_ref, out_ref,
                    p_ref):
    i = pl.program_id(0)

    @pl.when(i == 0)
    def _():
        st = st_ref[...]
        s1 = jnp.sum(st[:, 0, :], axis=0)
        s2 = jnp.sum(st[:, 1, :], axis=0)
        m = s1 / n_true
        v = s2 / n_true - m * m
        p_ref[0, :] = m
        p_ref[1, :] = g_ref[0, :] * lax.rsqrt(v + EPS)

    zh = (z_ref[...] - p_ref[0:1, :]) * p_ref[1:2, :] + be_ref[...]
    h = jnp.where(zh >= 0, zh, 0.2 * zh)
    out_ref[...] = lax.dot_general(
        h, w_ref[...], (((1,), (1,)), ((), ())),
        preferred_element_type=jnp.float32)


def _tc_y_norm(z, st, g, be, w_all, n_pad, n_true, c, c_out7):
    nb = n_pad // BN
    return pl.pallas_call(
        functools.partial(_tc_y_norm_body, float(n_true)),
        grid=(nb,),
        in_specs=[
            pl.BlockSpec((BN, c), lambda i: (i, 0)),
            pl.BlockSpec((NW, 2, c), lambda i: (0, 0, 0)),
            pl.BlockSpec((1, c), lambda i: (0, 0)),
            pl.BlockSpec((1, c), lambda i: (0, 0)),
            pl.BlockSpec(w_all.shape, lambda i: (0, 0)),
        ],
        out_specs=pl.BlockSpec((BN, c_out7), lambda i: (i, 0)),
        out_shape=jax.ShapeDtypeStruct((n_pad, c_out7), jnp.float32),
        scratch_shapes=[pltpu.VMEM((2, c), jnp.float32)],
    )(z, st, g, be, w_all)


def _tc_params_body(n_true, st_ref, g_ref, be_ref, out_ref):
    st = st_ref[...]
    s1 = jnp.sum(st[:, 0, :], axis=0)
    s2 = jnp.sum(st[:, 1, :], axis=0)
    m = s1 / n_true
    v = s2 / n_true - m * m
    out_ref[0, :] = m
    out_ref[1, :] = g_ref[0, :] * lax.rsqrt(v + EPS)
    out_ref[2, :] = be_ref[0, :]


def _tc_params(st, g, be, n_true, c):
    return pl.pallas_call(
        functools.partial(_tc_params_body, float(n_true)),
        out_shape=jax.ShapeDtypeStruct((3, c), jnp.float32),
    )(st, g, be)


def _tc_final_body(n_true, nb, z_ref, st_ref, g_ref, be_ref, wfc_ref, bfc_ref,
                   out_ref, p_ref, acc_ref):
    i = pl.program_id(0)

    @pl.when(i == 0)
    def _():
        st = st_ref[...]
        s1 = jnp.sum(st[:, 0, :], axis=0)
        s2 = jnp.sum(st[:, 1, :], axis=0)
        m = s1 / n_true
        v = s2 / n_true - m * m
        p_ref[0, :] = m
        p_ref[1, :] = g_ref[0, :] * lax.rsqrt(v + EPS)
        acc_ref[...] = jnp.zeros_like(acc_ref)

    zh = (z_ref[...] - p_ref[0:1, :]) * p_ref[1:2, :] + be_ref[...]
    h = jnp.where(zh >= 0, zh, 0.2 * zh)
    gid = i * BN + lax.broadcasted_iota(jnp.int32, (BN, 1), 0)
    h = jnp.where(gid < jnp.int32(n_true), h, 0.0)
    acc_ref[...] += jnp.sum(h, axis=0, keepdims=True)

    @pl.when(i == nb - 1)
    def _():
        mean = acc_ref[...] / n_true
        out_ref[...] = lax.dot_general(
            mean, wfc_ref[...], (((1,), (1,)), ((), ())),
            preferred_element_type=jnp.float32) + bfc_ref[...]


def _tc_final(z, st, g, be, wfc, bfc, n_pad, n_true, c):
    nb = n_pad // BN
    return pl.pallas_call(
        functools.partial(_tc_final_body, float(n_true), nb),
        grid=(nb,),
        in_specs=[
            pl.BlockSpec((BN, c), lambda i: (i, 0)),
            pl.BlockSpec((NW, 2, c), lambda i: (0, 0, 0)),
            pl.BlockSpec((1, c), lambda i: (0, 0)),
            pl.BlockSpec((1, c), lambda i: (0, 0)),
            pl.BlockSpec(wfc.shape, lambda i: (0, 0)),
            pl.BlockSpec(bfc.shape, lambda i: (0, 0)),
        ],
        out_specs=pl.BlockSpec((1, 36), lambda i: (0, 0)),
        out_shape=jax.ShapeDtypeStruct((1, 36), jnp.float32),
        scratch_shapes=[pltpu.VMEM((2, c), jnp.float32),
                        pltpu.VMEM((1, c), jnp.float32)],
    )(z, st, g, be, wfc, bfc)


def _make_sc_accum(n_pad, rw, b, c_out, n_true):
    nch = rw // b
    cv = c_out // 16
    mesh = plsc.VectorSubcoreMesh(core_axis_name="c", subcore_axis_name="s",
                                  num_cores=2, num_subcores=16)

    @functools.partial(
        pl.kernel,
        out_type=[jax.ShapeDtypeStruct((n_pad, c_out), jnp.float32),
                  jax.ShapeDtypeStruct((NW, 2, c_out), jnp.float32)],
        mesh=mesh,
        compiler_params=pltpu.CompilerParams(use_tc_tiling_on_sc=False),
        scratch_types=[pltpu.VMEM((rw,), jnp.int32) for _ in range(7)]
        + [pltpu.VMEM((b, c_out), jnp.float32) for _ in range(7)]
        + [pltpu.VMEM((b, c_out), jnp.float32),
           pltpu.VMEM((2, c_out), jnp.float32),
           pltpu.SemaphoreType.DMA],
    )
    def k(y_hbm, idx_hbm, z_hbm, st_hbm, i0, i1, i2, i3, i4, i5, i6,
          b0, b1, b2, b3, b4, b5, b6, zbuf, stbuf, gsem):
        idxs = [i0, i1, i2, i3, i4, i5, i6]
        bufs = [b0, b1, b2, b3, b4, b5, b6]
        wid = lax.axis_index("s") * 2 + lax.axis_index("c")
        base = wid * rw
        for j in range(7):
            pltpu.sync_copy(idx_hbm.at[pl.ds(j * n_pad + base, rw)], idxs[j])

        def chunk_body(ci, st):
            off = ci * b
            cps = [pltpu.async_copy(
                y_hbm.at[idxs[j].at[pl.ds(off, b)]], bufs[j], gsem)
                for j in range(7)]
            for cp in cps:
                cp.wait()

            def row_body(r, st):
                row = base + off + r
                valid = row < n_true
                new = list(st)
                for c in range(cv):
                    s = pl.ds(c * 16, 16)
                    zc = bufs[0][r, s]
                    for j in range(1, 7):
                        zc = zc + bufs[j][r, s]
                    zbuf[r, s] = zc
                    zm = jnp.where(valid, zc, 0.0)
                    new[c] = st[c] + zm
                    new[cv + c] = st[cv + c] + zm * zm
                return tuple(new)

            st = lax.fori_loop(0, b, row_body, st)
            pltpu.sync_copy(zbuf, z_hbm.at[pl.ds(base + off, b)])
            return st

        st0 = tuple(jnp.zeros((16,), jnp.float32) for _ in range(2 * cv))
        st = lax.fori_loop(0, nch, chunk_body, st0)
        for c in range(cv):
            s = pl.ds(c * 16, 16)
            stbuf[0, s] = st[c]
            stbuf[1, s] = st[cv + c]
        pltpu.sync_copy(stbuf, st_hbm.at[wid])

    return k


def _make_sc_pool(rw, b):
    nch = rw // b
    cv = 2
    mesh = plsc.VectorSubcoreMesh(core_axis_name="c", subcore_axis_name="s",
                                  num_cores=2, num_subcores=16)

    @functools.partial(
        pl.kernel,
        out_type=jax.ShapeDtypeStruct((N1P, 32), jnp.float32),
        mesh=mesh,
        compiler_params=pltpu.CompilerParams(use_tc_tiling_on_sc=False),
        scratch_types=[pltpu.VMEM((rw,), jnp.int32) for _ in range(7)]
        + [pltpu.VMEM((3, 32), jnp.float32)]
        + [pltpu.VMEM((b, 32), jnp.float32) for _ in range(7)]
        + [pltpu.VMEM((b, 32), jnp.float32),
           pltpu.SemaphoreType.DMA],
    )
    def k(z_hbm, idx_hbm, par_hbm, p_hbm, i0, i1, i2, i3, i4, i5, i6,
          par_v, b0, b1, b2, b3, b4, b5, b6, pbuf, gsem):
        idxs = [i0, i1, i2, i3, i4, i5, i6]
        bufs = [b0, b1, b2, b3, b4, b5, b6]
        wid = lax.axis_index("s") * 2 + lax.axis_index("c")
        base = wid * rw
        pltpu.sync_copy(par_hbm, par_v)
        for j in range(7):
            pltpu.sync_copy(idx_hbm.at[pl.ds(j * N1P + base, rw)], idxs[j])

        def chunk_body(ci, _):
            off = ci * b
            cps = [pltpu.async_copy(
                z_hbm.at[idxs[j].at[pl.ds(off, b)]], bufs[j], gsem)
                for j in range(7)]
            for cp in cps:
                cp.wait()

            def row_body(r, _):
                for c in range(cv):
                    s = pl.ds(c * 16, 16)
                    m = par_v[0, s]
                    sc = par_v[1, s]
                    be = par_v[2, s]
                    acc = jnp.zeros((16,), jnp.float32)
                    for j in range(7):
                        zh = (bufs[j][r, s] - m) * sc + be
                        acc = acc + jnp.where(zh >= 0, zh, 0.2 * zh)
                    pbuf[r, s] = acc * (1.0 / 7.0)
                return 0

            lax.fori_loop(0, b, row_body, 0)
            pltpu.sync_copy(pbuf, p_hbm.at[pl.ds(base + off, b)])
            return 0

        lax.fori_loop(0, nch, chunk_body, 0)

    return k


def _stack_w(w, c_in, c_out):
    return w.reshape(c_out, 7, c_in).transpose(1, 0, 2).reshape(
        7 * c_out, c_in)


def kernel(x, no0, no1, W1, b1, g1, be1, W2, b2, g2, be2, W3, b3, g3, be3,
           W4, b4, g4, be4, W5, b5, g5, be5, Wfc, bfc):
    f32 = jnp.float32
    ar7 = jnp.arange(7, dtype=jnp.int32)
    no0m = no0.reshape(N0, 7)
    no1m = no1.reshape(N1, 7)
    idxT0 = jnp.zeros((7, N0P), jnp.int32).at[:, :N0].set(
        (no0m * 7 + ar7).T).reshape(-1)
    idxT1 = jnp.zeros((7, N1P), jnp.int32).at[:, :N1].set(
        (no1m * 7 + ar7).T).reshape(-1)
    idxP = jnp.zeros((7, N1P), jnp.int32).at[:, :N1].set(
        no0m[:N1].T).reshape(-1)

    Wa1 = _stack_w(W1, 3, 32)
    Wa2 = _stack_w(W2, 32, 32)
    Wa3 = _stack_w(W3, 32, 32)
    Wa4 = _stack_w(W4, 32, 64)
    Wa5 = _stack_w(W5, 64, 64)
    g1r, be1r = g1.reshape(1, 32), be1.reshape(1, 32)
    g2r, be2r = g2.reshape(1, 32), be2.reshape(1, 32)
    g3r, be3r = g3.reshape(1, 32), be3.reshape(1, 32)
    g4r, be4r = g4.reshape(1, 64), be4.reshape(1, 64)
    g5r, be5r = g5.reshape(1, 64), be5.reshape(1, 64)
    bfcr = bfc.reshape(1, 36)

    xp = jnp.zeros((N0P, 3), f32).at[:N0].set(x)

    sc_acc0 = _make_sc_accum(N0P, RW0, B0, 32, N0)
    sc_acc1 = _make_sc_accum(N1P, RW1, B1, 64, N1)
    sc_pool = _make_sc_pool(RW1, B1)

    Y = _tc_y_plain(xp, Wa1, N0P, 3, 224).reshape(N0P * 7, 32)
    z1, st1 = sc_acc0(Y, idxT0)
    Y = _tc_y_norm(z1, st1, g1r, be1r, Wa2, N0P, N0, 32, 224)
    z2, st2 = sc_acc0(Y.reshape(N0P * 7, 32), idxT0)
    Y = _tc_y_norm(z2, st2, g2r, be2r, Wa3, N0P, N0, 32, 224)
    z3, st3 = sc_acc0(Y.reshape(N0P * 7, 32), idxT0)
    par3 = _tc_params(st3, g3r, be3r, N0, 32)
    p = sc_pool(z3, idxP, par3)
    Y = _tc_y_plain(p, Wa4, N1P, 32, 448).reshape(N1P * 7, 64)
    z4, st4 = sc_acc1(Y, idxT1)
    Y = _tc_y_norm(z4, st4, g4r, be4r, Wa5, N1P, N1, 64, 448)
    z5, st5 = sc_acc1(Y.reshape(N1P * 7, 64), idxT1)
    return _tc_final(z5, st5, g5r, be5r, Wfc, bfcr, N1P, N1, 64)
